# baseline (speedup 1.0000x reference)
import jax
import jax.numpy as jnp
from jax import lax
from jax.experimental import pallas as pl
from jax.experimental.pallas import tpu as pltpu
from jax.experimental.pallas import tpu_sc as plsc

_B = 4096
_L = 200
_LP = 208
_DIM = 64
_PW = _DIM // 2
_NTILES = 32
_UPT = _B // _NTILES
_NG = _LP // 16
_RING = 8


def _issue_user(ip_hbm, ib_hbm, mask_hbm, ids_v, rows_v, ibias_v, mask_v,
                sem, i, bg):
    idx_a = ids_v.at[i, pl.ds(0, 128)]
    idx_b = ids_v.at[i, pl.ds(128, 72)]
    pltpu.async_copy(ip_hbm.at[idx_a], rows_v.at[pl.ds(0, 128)], sem)
    pltpu.async_copy(ip_hbm.at[idx_b], rows_v.at[pl.ds(128, 72)], sem)
    pltpu.async_copy(ib_hbm.at[idx_a], ibias_v.at[pl.ds(0, 128)], sem)
    pltpu.async_copy(ib_hbm.at[idx_b], ibias_v.at[pl.ds(128, 72)], sem)
    pltpu.async_copy(mask_hbm.at[bg], mask_v.at[pl.ds(0, _L)], sem)


def _wait_user(ip_hbm, ib_hbm, mask_hbm, rows_v, ibias_v, mask_v, sem, bg):
    pltpu.make_async_copy(ip_hbm.at[pl.ds(0, 128)],
                          rows_v.at[pl.ds(0, 128)], sem).wait()
    pltpu.make_async_copy(ip_hbm.at[pl.ds(0, 72)],
                          rows_v.at[pl.ds(128, 72)], sem).wait()
    pltpu.make_async_copy(ib_hbm.at[pl.ds(0, 128)],
                          ibias_v.at[pl.ds(0, 128)], sem).wait()
    pltpu.make_async_copy(ib_hbm.at[pl.ds(0, 72)],
                          ibias_v.at[pl.ds(128, 72)], sem).wait()
    pltpu.make_async_copy(mask_hbm.at[bg], mask_v.at[pl.ds(0, _L)],
                          sem).wait()


def _tec_body(uf_hbm, ip_hbm, ub_hbm, ib_hbm, user_hbm, item_hbm, mask_hbm,
              out_hbm,
              uidx_v, ufac_v, ubias_v, ids_v, ufac_s,
              rows_r, ibias_r, mask_r, out_r,
              sems, sems_out, sem_misc):
    nc = 2
    wid = lax.axis_index("s") * nc + lax.axis_index("c")
    base = wid * _UPT

    pltpu.sync_copy(user_hbm.at[pl.ds(base, _UPT)], uidx_v)
    pltpu.sync_copy(item_hbm.at[pl.ds(base, _UPT)], ids_v)
    pltpu.async_copy(uf_hbm.at[uidx_v], ufac_v, sem_misc).wait()
    pltpu.async_copy(ub_hbm.at[uidx_v], ubias_v.at[pl.ds(0, _UPT)],
                     sem_misc).wait()

    for b in range(_RING):
        _issue_user(ip_hbm, ib_hbm, mask_hbm, ids_v, rows_r[b], ibias_r[b],
                    mask_r[b], sems[b], b, base + b)

    @pl.loop(0, _UPT, step=_RING)
    def _outer(p):
        for b in range(_RING):
            i = p + b
            bg = base + i
            rows_v, ibias_v, mask_v, out_v = (
                rows_r[b], ibias_r[b], mask_r[b], out_r[b])
            _wait_user(ip_hbm, ib_hbm, mask_hbm, rows_v, ibias_v, mask_v,
                       sems[b], bg)

            @pl.when(p > 0)
            def _drain_out():
                pltpu.make_async_copy(out_v.at[pl.ds(0, _L)],
                                      out_hbm.at[bg], sems_out[b]).wait()

            ub = ubias_v[pl.ds(i, 16)][0]
            for k in range(4):
                uv = ufac_v[i, pl.ds(16 * k, 16)]
                for j in range(16):
                    ufac_s[16 * k + j] = uv[j]

            @pl.loop(0, _NG)
            def _group(g):
                off = pl.multiple_of(g * 16, 16)
                row_ids = lax.iota(jnp.int32, 16) + g * 16
                acc = ibias_v[pl.ds(off, 16)] + ub
                cols = jnp.zeros((16,), jnp.int32)
                for hp in range(_PW):
                    w = plsc.load_gather(rows_v, [row_ids, cols])
                    pair = plsc.bitcast(w, jnp.bfloat16)
                    va, vb = plsc.unpack(
                        pair, format=plsc.PackFormat.INTERLEAVED)
                    acc = acc + va * ufac_s[2 * hp] + vb * ufac_s[2 * hp + 1]
                    cols = cols + 1
                m = mask_v[pl.ds(off, 16)]
                out_v[pl.ds(off, 16)] = jnp.where(
                    m == 0, jnp.float32(-1e13), acc)

            pltpu.async_copy(out_v.at[pl.ds(0, _L)], out_hbm.at[bg],
                             sems_out[b])

            @pl.when(i + _RING < _UPT)
            def _issue_next():
                _issue_user(ip_hbm, ib_hbm, mask_hbm, ids_v, rows_v, ibias_v,
                            mask_v, sems[b], i + _RING, bg + _RING)

    for b in range(_RING):
        pltpu.make_async_copy(out_r[b].at[pl.ds(0, _L)],
                              out_hbm.at[base], sems_out[b]).wait()


@jax.jit
def kernel(user_factors, item_factors, user_biases, item_biases,
           user, item, mask):
    packed = lax.bitcast_convert_type(
        item_factors.astype(jnp.bfloat16).reshape(-1, _PW, 2), jnp.int32)

    mesh = plsc.VectorSubcoreMesh(core_axis_name="c", subcore_axis_name="s")
    run = pl.kernel(
        _tec_body,
        out_type=jax.ShapeDtypeStruct((_B, _L), jnp.float32),
        mesh=mesh,
        scratch_types=[
            pltpu.VMEM((_UPT,), jnp.int32),
            pltpu.VMEM((_UPT, _DIM), jnp.float32),
            pltpu.VMEM((_UPT + 16,), jnp.float32),
            pltpu.VMEM((_UPT, _L), jnp.int32),
            pltpu.SMEM((_DIM,), jnp.float32),
            [pltpu.VMEM((_LP, _PW), jnp.int32)] * _RING,
            [pltpu.VMEM((_LP,), jnp.float32)] * _RING,
            [pltpu.VMEM((_LP,), jnp.int32)] * _RING,
            [pltpu.VMEM((_LP,), jnp.float32)] * _RING,
            [pltpu.SemaphoreType.DMA] * _RING,
            [pltpu.SemaphoreType.DMA] * _RING,
            pltpu.SemaphoreType.DMA,
        ],
        compiler_params=pltpu.CompilerParams(
            needs_layout_passes=False, use_tc_tiling_on_sc=False),
    )
    return run(user_factors, packed, user_biases, item_biases,
               user.astype(jnp.int32), item.astype(jnp.int32), mask)


# speedup vs baseline: 1.5693x; 1.5693x over previous
import jax
import jax.numpy as jnp
from jax import lax
from jax.experimental import pallas as pl
from jax.experimental.pallas import tpu as pltpu
from jax.experimental.pallas import tpu_sc as plsc

_B = 4096
_L = 200
_DIM = 64
_NTILES = 32
_UPT = _B // _NTILES
_UPC = 2
_CL = _UPC * _L
_CLP = _CL + 8
_NC = _UPT // _UPC
_NG = 13
_RING = 2


def _issue_chunk(if_hbm, ib_hbm, mask_hbm, ids_v, rows_v, ibias_v, mask_v,
                 sem, c, fb):
    idx = ids_v.at[pl.ds(c * _CL, _CL)]
    pltpu.async_copy(if_hbm.at[idx], rows_v.at[pl.ds(0, _CL)], sem)
    pltpu.async_copy(ib_hbm.at[idx], ibias_v.at[pl.ds(0, _CL)], sem)
    pltpu.async_copy(mask_hbm.at[pl.ds(fb, _CL)], mask_v.at[pl.ds(0, _CL)],
                     sem)


def _wait_chunk(if_hbm, ib_hbm, mask_hbm, rows_v, ibias_v, mask_v, sem):
    pltpu.make_async_copy(if_hbm.at[pl.ds(0, _CL)],
                          rows_v.at[pl.ds(0, _CL)], sem).wait()
    pltpu.make_async_copy(ib_hbm.at[pl.ds(0, _CL)],
                          ibias_v.at[pl.ds(0, _CL)], sem).wait()
    pltpu.make_async_copy(mask_hbm.at[pl.ds(0, _CL)],
                          mask_v.at[pl.ds(0, _CL)], sem).wait()


def _tec_body(uf_hbm, if_hbm, ub_hbm, ib_hbm, user_hbm, item_hbm, mask_hbm,
              out_hbm,
              uidx_v, ufac_v, ubias_v, ids_v, ufac_s,
              rows_r, ibias_r, mask_r, out_r,
              sems, sems_out, sem_misc):
    nc = 2
    wid = lax.axis_index("s") * nc + lax.axis_index("c")
    base = wid * _UPT
    fbase = base * _L

    pltpu.sync_copy(user_hbm.at[pl.ds(base, _UPT)], uidx_v)
    pltpu.sync_copy(item_hbm.at[pl.ds(fbase, _UPT * _L)], ids_v)
    pltpu.async_copy(uf_hbm.at[uidx_v], ufac_v, sem_misc).wait()
    pltpu.async_copy(ub_hbm.at[uidx_v], ubias_v.at[pl.ds(0, _UPT)],
                     sem_misc).wait()

    for s in range(_RING):
        _issue_chunk(if_hbm, ib_hbm, mask_hbm, ids_v, rows_r[s], ibias_r[s],
                     mask_r[s], sems[s], s, fbase + s * _CL)

    @pl.loop(0, _NC, step=_RING)
    def _outer(p):
        for s in range(_RING):
            c = p + s
            rows_v, ibias_v, mask_v, out_v = (
                rows_r[s], ibias_r[s], mask_r[s], out_r[s])
            _wait_chunk(if_hbm, ib_hbm, mask_hbm, rows_v, ibias_v, mask_v,
                        sems[s])

            @pl.when(p > 0)
            def _drain_out():
                pltpu.make_async_copy(out_v.at[pl.ds(0, _CL)],
                                      out_hbm.at[pl.ds(0, _CL)],
                                      sems_out[s]).wait()

            for j in range(_UPC):
                i = c * _UPC + j
                ub = ubias_v[pl.ds(i, 16)][0]
                for k in range(4):
                    uv = ufac_v[i, pl.ds(16 * k, 16)]
                    for t in range(16):
                        ufac_s[16 * k + t] = uv[t]

                jo = j * _L

                @pl.loop(0, _NG)
                def _group(g):
                    off = jo + g * 16
                    row_ids = lax.iota(jnp.int32, 16) + off
                    acc = ibias_v[pl.ds(off, 16)] + ub
                    cols = jnp.zeros((16,), jnp.int32)
                    for h in range(_DIM):
                        vals = plsc.load_gather(rows_v, [row_ids, cols])
                        acc = acc + vals * ufac_s[h]
                        cols = cols + 1
                    m = mask_v[pl.ds(off, 16)]
                    out_v[pl.ds(off, 16)] = jnp.where(
                        m == 0, jnp.float32(-1e13), acc)

            pltpu.async_copy(out_v.at[pl.ds(0, _CL)],
                             out_hbm.at[pl.ds(fbase + c * _CL, _CL)],
                             sems_out[s])

            @pl.when(c + _RING < _NC)
            def _issue_next():
                _issue_chunk(if_hbm, ib_hbm, mask_hbm, ids_v, rows_v,
                             ibias_v, mask_v, sems[s],
                             c + _RING, fbase + (c + _RING) * _CL)

    for s in range(_RING):
        pltpu.make_async_copy(out_r[s].at[pl.ds(0, _CL)],
                              out_hbm.at[pl.ds(0, _CL)], sems_out[s]).wait()


@jax.jit
def kernel(user_factors, item_factors, user_biases, item_biases,
           user, item, mask):
    mesh = plsc.VectorSubcoreMesh(core_axis_name="c", subcore_axis_name="s")
    run = pl.kernel(
        _tec_body,
        out_type=jax.ShapeDtypeStruct((_B * _L,), jnp.float32),
        mesh=mesh,
        scratch_types=[
            pltpu.VMEM((_UPT,), jnp.int32),
            pltpu.VMEM((_UPT, _DIM), jnp.float32),
            pltpu.VMEM((_UPT + 16,), jnp.float32),
            pltpu.VMEM((_UPT * _L,), jnp.int32),
            pltpu.SMEM((_DIM,), jnp.float32),
            [pltpu.VMEM((_CLP, _DIM), jnp.float32)] * _RING,
            [pltpu.VMEM((_CLP,), jnp.float32)] * _RING,
            [pltpu.VMEM((_CLP,), jnp.int32)] * _RING,
            [pltpu.VMEM((_CLP,), jnp.float32)] * _RING,
            [pltpu.SemaphoreType.DMA] * _RING,
            [pltpu.SemaphoreType.DMA] * _RING,
            pltpu.SemaphoreType.DMA,
        ],
        compiler_params=pltpu.CompilerParams(
            needs_layout_passes=False, use_tc_tiling_on_sc=False),
    )
    out = run(user_factors, item_factors, user_biases, item_biases,
              user.astype(jnp.int32), item.reshape(-1).astype(jnp.int32),
              mask.reshape(-1))
    return out.reshape(_B, _L)
